# baseline (speedup 1.0000x reference)
import functools

import jax
import jax.numpy as jnp
from jax import lax
from jax.experimental import pallas as pl
from jax.experimental.pallas import tpu as pltpu
from jax.experimental.pallas import tpu_sc as plsc

EMB_DIM = 128
PART = EMB_DIM // 3
OFF_DIM = EMB_DIM - 2 * PART

B = 16384
NC, NS, LANES = 2, 16, 16
NW = NC * NS
ROWS_PER_W = B // NW
CHUNK = 128
NCH = ROWS_PER_W // CHUNK
N_FUSED = 2 * 7 * 200


def _sc_embed(idx_s, idx_d, idx_o, fused_table):
    mesh = plsc.VectorSubcoreMesh(core_axis_name="c", subcore_axis_name="s")

    @functools.partial(
        pl.kernel,
        out_type=jax.ShapeDtypeStruct((B, EMB_DIM), jnp.float32),
        mesh=mesh,
        scratch_types=[
            pltpu.VMEM((ROWS_PER_W,), jnp.int32),
            pltpu.VMEM((NCH, CHUNK, EMB_DIM), jnp.float32),
            pltpu.SemaphoreType.DMA,
            pltpu.SemaphoreType.DMA,
            pltpu.SemaphoreType.DMA,
        ],
    )
    def body(idx_f_hbm, ft_hbm, out_hbm,
             idx_fv, comb, sem_i, sem_g, sem_w):
        wid = lax.axis_index("s") * NC + lax.axis_index("c")
        base = wid * ROWS_PER_W
        pltpu.async_copy(idx_f_hbm.at[pl.ds(base, ROWS_PER_W)], idx_fv,
                         sem_i).wait()
        gs = [pltpu.async_copy(ft_hbm.at[idx_fv.at[pl.ds(j * CHUNK, CHUNK)]],
                               comb.at[j], sem_g)
              for j in range(NCH)]
        ws = []
        for j in range(NCH):
            gs[j].wait()
            ws.append(pltpu.async_copy(
                comb.at[j], out_hbm.at[pl.ds(base + j * CHUNK, CHUNK), :],
                sem_w))
        for w in ws:
            w.wait()

    return body(idx_s * (7 * 200) + idx_d * 200 + idx_o, fused_table)


@jax.jit
def kernel(scale_conditions, distortion_conditions, offset_conditions,
           scale_table, distortion_table, offset_table):
    idx_s = scale_conditions.astype(jnp.int32)
    idx_d = distortion_conditions.astype(jnp.int32)
    idx_o = offset_conditions.astype(jnp.int32)
    s_b = jnp.broadcast_to(scale_table[:, None, None, :], (2, 7, 200, PART))
    d_b = jnp.broadcast_to(distortion_table[None, :, None, :], (2, 7, 200, PART))
    o_b = jnp.broadcast_to(offset_table[None, None, :, :], (2, 7, 200, OFF_DIM))
    fused_table = jnp.concatenate([s_b, d_b, o_b], axis=-1).reshape(
        N_FUSED, EMB_DIM)
    return _sc_embed(idx_s, idx_d, idx_o, fused_table)


# speedup vs baseline: 1.0585x; 1.0585x over previous
import functools

import jax
import jax.numpy as jnp
from jax import lax
from jax.experimental import pallas as pl
from jax.experimental.pallas import tpu as pltpu
from jax.experimental.pallas import tpu_sc as plsc

EMB_DIM = 128
PART = EMB_DIM // 3
OFF_DIM = EMB_DIM - 2 * PART

B = 16384
NC, NS, LANES = 2, 16, 16
NW = NC * NS
ROWS_PER_W = B // NW
CHUNK = 128
NCH = ROWS_PER_W // CHUNK
N_FUSED = 2 * 7 * 200


def _sc_embed(idx_s, idx_d, idx_o, fused_table):
    mesh = plsc.VectorSubcoreMesh(core_axis_name="c", subcore_axis_name="s")

    @functools.partial(
        pl.kernel,
        out_type=jax.ShapeDtypeStruct((B, EMB_DIM), jnp.float32),
        mesh=mesh,
        scratch_types=[
            pltpu.VMEM((ROWS_PER_W,), jnp.int32),
            pltpu.VMEM((ROWS_PER_W,), jnp.int32),
            pltpu.VMEM((ROWS_PER_W,), jnp.int32),
            pltpu.VMEM((ROWS_PER_W,), jnp.int32),
            pltpu.VMEM((NCH, CHUNK, EMB_DIM), jnp.float32),
            pltpu.SemaphoreType.DMA,
            pltpu.SemaphoreType.DMA,
            pltpu.SemaphoreType.DMA,
        ],
    )
    def body(idx_s_hbm, idx_d_hbm, idx_o_hbm, ft_hbm, out_hbm,
             idx_sv, idx_dv, idx_ov, idx_fv, comb, sem_i, sem_g, sem_w):
        wid = lax.axis_index("s") * NC + lax.axis_index("c")
        base = wid * ROWS_PER_W
        ci = [pltpu.async_copy(idx_s_hbm.at[pl.ds(base, ROWS_PER_W)], idx_sv,
                               sem_i),
              pltpu.async_copy(idx_d_hbm.at[pl.ds(base, ROWS_PER_W)], idx_dv,
                               sem_i),
              pltpu.async_copy(idx_o_hbm.at[pl.ds(base, ROWS_PER_W)], idx_ov,
                               sem_i)]
        for c in ci:
            c.wait()
        def fuse(k, carry):
            sl = pl.ds(pl.multiple_of(k * LANES, LANES), LANES)
            idx_fv[sl] = (idx_sv[sl] * (7 * 200) + idx_dv[sl] * 200) + idx_ov[sl]
            return carry

        lax.fori_loop(0, ROWS_PER_W // LANES, fuse, 0)
        gs = [pltpu.async_copy(ft_hbm.at[idx_fv.at[pl.ds(j * CHUNK, CHUNK)]],
                               comb.at[j], sem_g)
              for j in range(NCH)]
        ws = []
        for j in range(NCH):
            gs[j].wait()
            ws.append(pltpu.async_copy(
                comb.at[j], out_hbm.at[pl.ds(base + j * CHUNK, CHUNK), :],
                sem_w))
        for w in ws:
            w.wait()

    return body(idx_s, idx_d, idx_o, fused_table)


@jax.jit
def kernel(scale_conditions, distortion_conditions, offset_conditions,
           scale_table, distortion_table, offset_table):
    idx_s = scale_conditions.astype(jnp.int32)
    idx_d = distortion_conditions.astype(jnp.int32)
    idx_o = offset_conditions.astype(jnp.int32)
    s_b = jnp.broadcast_to(scale_table[:, None, None, :], (2, 7, 200, PART))
    d_b = jnp.broadcast_to(distortion_table[None, :, None, :], (2, 7, 200, PART))
    o_b = jnp.broadcast_to(offset_table[None, None, :, :], (2, 7, 200, OFF_DIM))
    fused_table = jnp.concatenate([s_b, d_b, o_b], axis=-1).reshape(
        N_FUSED, EMB_DIM)
    return _sc_embed(idx_s, idx_d, idx_o, fused_table)


# speedup vs baseline: 1.1316x; 1.0691x over previous
import functools

import jax
import jax.numpy as jnp
from jax import lax
from jax.experimental import pallas as pl
from jax.experimental.pallas import tpu as pltpu
from jax.experimental.pallas import tpu_sc as plsc

EMB_DIM = 128
PART = EMB_DIM // 3
OFF_DIM = EMB_DIM - 2 * PART

B = 16384
NC, NS, LANES = 2, 16, 16
NW = NC * NS
ROWS_PER_W = B // NW
CHUNK = 128
NCH = ROWS_PER_W // CHUNK
N_FUSED = 2 * 7 * 200
N_FUSED_PAD = 2816


def _sc_embed(idx_s, idx_d, idx_o, fused_table):
    mesh = plsc.VectorSubcoreMesh(core_axis_name="c", subcore_axis_name="s")

    @functools.partial(
        pl.kernel,
        out_type=jax.ShapeDtypeStruct((B, EMB_DIM), jnp.float32),
        mesh=mesh,
        scratch_types=[
            pltpu.VMEM((ROWS_PER_W,), jnp.int32),
            pltpu.VMEM((ROWS_PER_W,), jnp.int32),
            pltpu.VMEM((ROWS_PER_W,), jnp.int32),
            pltpu.VMEM((ROWS_PER_W,), jnp.int32),
            pltpu.VMEM((NCH, CHUNK, EMB_DIM), jnp.float32),
            pltpu.VMEM_SHARED((N_FUSED_PAD, EMB_DIM), jnp.float32),
            pltpu.SemaphoreType.DMA,
            pltpu.SemaphoreType.DMA,
            pltpu.SemaphoreType.DMA,
        ],
    )
    def body(idx_s_hbm, idx_d_hbm, idx_o_hbm, ft_hbm, out_hbm,
             idx_sv, idx_dv, idx_ov, idx_fv, comb, sft, sem_i, sem_g, sem_w):
        sid = lax.axis_index("s")
        wid = sid * NC + lax.axis_index("c")
        base = wid * ROWS_PER_W
        st = pltpu.async_copy(ft_hbm.at[pl.ds(sid * 176, 176)],
                              sft.at[pl.ds(sid * 176, 176)], sem_g)
        ci = [pltpu.async_copy(idx_s_hbm.at[pl.ds(base, ROWS_PER_W)], idx_sv,
                               sem_i),
              pltpu.async_copy(idx_d_hbm.at[pl.ds(base, ROWS_PER_W)], idx_dv,
                               sem_i),
              pltpu.async_copy(idx_o_hbm.at[pl.ds(base, ROWS_PER_W)], idx_ov,
                               sem_i)]
        for c in ci:
            c.wait()
        def fuse(k, carry):
            sl = pl.ds(pl.multiple_of(k * LANES, LANES), LANES)
            idx_fv[sl] = (idx_sv[sl] * (7 * 200) + idx_dv[sl] * 200) + idx_ov[sl]
            return carry

        lax.fori_loop(0, ROWS_PER_W // LANES, fuse, 0)
        st.wait()
        plsc.subcore_barrier()
        gs = [pltpu.async_copy(sft.at[idx_fv.at[pl.ds(j * CHUNK, CHUNK)]],
                               comb.at[j], sem_g)
              for j in range(NCH)]
        ws = []
        for j in range(NCH):
            gs[j].wait()
            ws.append(pltpu.async_copy(
                comb.at[j], out_hbm.at[pl.ds(base + j * CHUNK, CHUNK), :],
                sem_w))
        for w in ws:
            w.wait()

    return body(idx_s, idx_d, idx_o, fused_table)


@jax.jit
def kernel(scale_conditions, distortion_conditions, offset_conditions,
           scale_table, distortion_table, offset_table):
    idx_s = scale_conditions.astype(jnp.int32)
    idx_d = distortion_conditions.astype(jnp.int32)
    idx_o = offset_conditions.astype(jnp.int32)
    s_b = jnp.broadcast_to(scale_table[:, None, None, :], (2, 7, 200, PART))
    d_b = jnp.broadcast_to(distortion_table[None, :, None, :], (2, 7, 200, PART))
    o_b = jnp.broadcast_to(offset_table[None, None, :, :], (2, 7, 200, OFF_DIM))
    fused_table = jnp.concatenate([s_b, d_b, o_b], axis=-1).reshape(
        N_FUSED, EMB_DIM)
    fused_table = jnp.pad(fused_table, ((0, N_FUSED_PAD - N_FUSED), (0, 0)))
    return _sc_embed(idx_s, idx_d, idx_o, fused_table)


# speedup vs baseline: 1.1416x; 1.0088x over previous
import functools

import jax
import jax.numpy as jnp
from jax import lax
from jax.experimental import pallas as pl
from jax.experimental.pallas import tpu as pltpu
from jax.experimental.pallas import tpu_sc as plsc

EMB_DIM = 128
PART = EMB_DIM // 3
OFF_DIM = EMB_DIM - 2 * PART

B = 16384
NC, NS, LANES = 2, 16, 16
NW = NC * NS
ROWS_PER_W = B // NW
CHUNK = 64
NCH = ROWS_PER_W // CHUNK
N_FUSED = 2 * 7 * 200
N_FUSED_PAD = 2816


def _sc_embed(idx_s, idx_d, idx_o, fused_table):
    mesh = plsc.VectorSubcoreMesh(core_axis_name="c", subcore_axis_name="s")

    @functools.partial(
        pl.kernel,
        out_type=jax.ShapeDtypeStruct((B, EMB_DIM), jnp.float32),
        mesh=mesh,
        scratch_types=[
            pltpu.VMEM((ROWS_PER_W,), jnp.int32),
            pltpu.VMEM((ROWS_PER_W,), jnp.int32),
            pltpu.VMEM((ROWS_PER_W,), jnp.int32),
            pltpu.VMEM((ROWS_PER_W,), jnp.int32),
            pltpu.VMEM((NCH, CHUNK, EMB_DIM), jnp.float32),
            pltpu.VMEM_SHARED((N_FUSED_PAD, EMB_DIM), jnp.float32),
            pltpu.SemaphoreType.DMA,
            pltpu.SemaphoreType.DMA,
            pltpu.SemaphoreType.DMA,
        ],
    )
    def body(idx_s_hbm, idx_d_hbm, idx_o_hbm, ft_hbm, out_hbm,
             idx_sv, idx_dv, idx_ov, idx_fv, comb, sft, sem_i, sem_g, sem_w):
        sid = lax.axis_index("s")
        wid = sid * NC + lax.axis_index("c")
        base = wid * ROWS_PER_W
        st = pltpu.async_copy(ft_hbm.at[pl.ds(sid * 176, 176)],
                              sft.at[pl.ds(sid * 176, 176)], sem_g)
        ci = [pltpu.async_copy(idx_s_hbm.at[pl.ds(base, ROWS_PER_W)], idx_sv,
                               sem_i),
              pltpu.async_copy(idx_d_hbm.at[pl.ds(base, ROWS_PER_W)], idx_dv,
                               sem_i),
              pltpu.async_copy(idx_o_hbm.at[pl.ds(base, ROWS_PER_W)], idx_ov,
                               sem_i)]
        for c in ci:
            c.wait()
        def fuse(k, carry):
            sl = pl.ds(pl.multiple_of(k * LANES, LANES), LANES)
            idx_fv[sl] = (idx_sv[sl] * (7 * 200) + idx_dv[sl] * 200) + idx_ov[sl]
            return carry

        lax.fori_loop(0, ROWS_PER_W // LANES, fuse, 0)
        st.wait()
        plsc.subcore_barrier()
        gs = [pltpu.async_copy(sft.at[idx_fv.at[pl.ds(j * CHUNK, CHUNK)]],
                               comb.at[j], sem_g)
              for j in range(NCH)]
        ws = []
        for j in range(NCH):
            gs[j].wait()
            ws.append(pltpu.async_copy(
                comb.at[j], out_hbm.at[pl.ds(base + j * CHUNK, CHUNK), :],
                sem_w))
        for w in ws:
            w.wait()

    return body(idx_s, idx_d, idx_o, fused_table)


@jax.jit
def kernel(scale_conditions, distortion_conditions, offset_conditions,
           scale_table, distortion_table, offset_table):
    idx_s = scale_conditions.astype(jnp.int32)
    idx_d = distortion_conditions.astype(jnp.int32)
    idx_o = offset_conditions.astype(jnp.int32)
    s_b = jnp.broadcast_to(scale_table[:, None, None, :], (2, 7, 200, PART))
    d_b = jnp.broadcast_to(distortion_table[None, :, None, :], (2, 7, 200, PART))
    o_b = jnp.broadcast_to(offset_table[None, None, :, :], (2, 7, 200, OFF_DIM))
    fused_table = jnp.concatenate([s_b, d_b, o_b], axis=-1).reshape(
        N_FUSED, EMB_DIM)
    fused_table = jnp.pad(fused_table, ((0, N_FUSED_PAD - N_FUSED), (0, 0)))
    return _sc_embed(idx_s, idx_d, idx_o, fused_table)


# speedup vs baseline: 1.1434x; 1.0015x over previous
import functools

import jax
import jax.numpy as jnp
from jax import lax
from jax.experimental import pallas as pl
from jax.experimental.pallas import tpu as pltpu
from jax.experimental.pallas import tpu_sc as plsc

EMB_DIM = 128
PART = EMB_DIM // 3
OFF_DIM = EMB_DIM - 2 * PART

B = 16384
NC, NS, LANES = 2, 16, 16
NW = NC * NS
ROWS_PER_W = B // NW
CHUNK = 32
NCH = ROWS_PER_W // CHUNK
N_FUSED = 2 * 7 * 200
N_FUSED_PAD = 2816


def _sc_embed(idx_s, idx_d, idx_o, fused_table):
    mesh = plsc.VectorSubcoreMesh(core_axis_name="c", subcore_axis_name="s")

    @functools.partial(
        pl.kernel,
        out_type=jax.ShapeDtypeStruct((B, EMB_DIM), jnp.float32),
        mesh=mesh,
        scratch_types=[
            pltpu.VMEM((ROWS_PER_W,), jnp.int32),
            pltpu.VMEM((ROWS_PER_W,), jnp.int32),
            pltpu.VMEM((ROWS_PER_W,), jnp.int32),
            pltpu.VMEM((ROWS_PER_W,), jnp.int32),
            pltpu.VMEM((NCH, CHUNK, EMB_DIM), jnp.float32),
            pltpu.VMEM_SHARED((N_FUSED_PAD, EMB_DIM), jnp.float32),
            pltpu.SemaphoreType.DMA,
            pltpu.SemaphoreType.DMA,
            pltpu.SemaphoreType.DMA,
        ],
    )
    def body(idx_s_hbm, idx_d_hbm, idx_o_hbm, ft_hbm, out_hbm,
             idx_sv, idx_dv, idx_ov, idx_fv, comb, sft, sem_i, sem_g, sem_w):
        sid = lax.axis_index("s")
        wid = sid * NC + lax.axis_index("c")
        base = wid * ROWS_PER_W
        st = pltpu.async_copy(ft_hbm.at[pl.ds(sid * 176, 176)],
                              sft.at[pl.ds(sid * 176, 176)], sem_g)
        ci = [pltpu.async_copy(idx_s_hbm.at[pl.ds(base, ROWS_PER_W)], idx_sv,
                               sem_i),
              pltpu.async_copy(idx_d_hbm.at[pl.ds(base, ROWS_PER_W)], idx_dv,
                               sem_i),
              pltpu.async_copy(idx_o_hbm.at[pl.ds(base, ROWS_PER_W)], idx_ov,
                               sem_i)]
        for c in ci:
            c.wait()
        def fuse(k, carry):
            sl = pl.ds(pl.multiple_of(k * LANES, LANES), LANES)
            idx_fv[sl] = (idx_sv[sl] * (7 * 200) + idx_dv[sl] * 200) + idx_ov[sl]
            return carry

        lax.fori_loop(0, ROWS_PER_W // LANES, fuse, 0)
        st.wait()
        plsc.subcore_barrier()
        gs = [pltpu.async_copy(sft.at[idx_fv.at[pl.ds(j * CHUNK, CHUNK)]],
                               comb.at[j], sem_g)
              for j in range(NCH)]
        ws = []
        for j in range(NCH):
            gs[j].wait()
            ws.append(pltpu.async_copy(
                comb.at[j], out_hbm.at[pl.ds(base + j * CHUNK, CHUNK), :],
                sem_w))
        for w in ws:
            w.wait()

    return body(idx_s, idx_d, idx_o, fused_table)


@jax.jit
def kernel(scale_conditions, distortion_conditions, offset_conditions,
           scale_table, distortion_table, offset_table):
    idx_s = scale_conditions.astype(jnp.int32)
    idx_d = distortion_conditions.astype(jnp.int32)
    idx_o = offset_conditions.astype(jnp.int32)
    s_b = jnp.broadcast_to(scale_table[:, None, None, :], (2, 7, 200, PART))
    d_b = jnp.broadcast_to(distortion_table[None, :, None, :], (2, 7, 200, PART))
    o_b = jnp.broadcast_to(offset_table[None, None, :, :], (2, 7, 200, OFF_DIM))
    fused_table = jnp.concatenate([s_b, d_b, o_b], axis=-1).reshape(
        N_FUSED, EMB_DIM)
    fused_table = jnp.pad(fused_table, ((0, N_FUSED_PAD - N_FUSED), (0, 0)))
    return _sc_embed(idx_s, idx_d, idx_o, fused_table)


# speedup vs baseline: 1.1551x; 1.0103x over previous
import functools

import jax
import jax.numpy as jnp
from jax import lax
from jax.experimental import pallas as pl
from jax.experimental.pallas import tpu as pltpu
from jax.experimental.pallas import tpu_sc as plsc

EMB_DIM = 128
PART = EMB_DIM // 3
OFF_DIM = EMB_DIM - 2 * PART

B = 16384
NC, NS, LANES = 2, 16, 16
NW = NC * NS
ROWS_PER_W = B // NW
CHUNK = 64
NCH = ROWS_PER_W // CHUNK
NCH_HBM = 3
N_FUSED = 2 * 7 * 200
N_FUSED_PAD = 2816
STAGE = N_FUSED_PAD // NS


def _sc_embed(idx_s, idx_d, idx_o, fused_table):
    mesh = plsc.VectorSubcoreMesh(core_axis_name="c", subcore_axis_name="s")

    @functools.partial(
        pl.kernel,
        out_type=jax.ShapeDtypeStruct((B, EMB_DIM), jnp.float32),
        mesh=mesh,
        scratch_types=[
            pltpu.VMEM((ROWS_PER_W,), jnp.int32),
            pltpu.VMEM((ROWS_PER_W,), jnp.int32),
            pltpu.VMEM((ROWS_PER_W,), jnp.int32),
            pltpu.VMEM((ROWS_PER_W,), jnp.int32),
            pltpu.VMEM((NCH, CHUNK, EMB_DIM), jnp.float32),
            pltpu.VMEM_SHARED((N_FUSED_PAD, EMB_DIM), jnp.float32),
            pltpu.SemaphoreType.DMA,
            pltpu.SemaphoreType.DMA,
            pltpu.SemaphoreType.DMA,
        ],
    )
    def body(idx_s_hbm, idx_d_hbm, idx_o_hbm, ft_hbm, out_hbm,
             idx_sv, idx_dv, idx_ov, idx_fv, comb, sft, sem_i, sem_g, sem_w):
        sid = lax.axis_index("s")
        wid = sid * NC + lax.axis_index("c")
        base = wid * ROWS_PER_W
        st = pltpu.async_copy(ft_hbm.at[pl.ds(sid * STAGE, STAGE)],
                              sft.at[pl.ds(sid * STAGE, STAGE)], sem_i)
        ci = [pltpu.async_copy(idx_s_hbm.at[pl.ds(base, ROWS_PER_W)], idx_sv,
                               sem_i),
              pltpu.async_copy(idx_d_hbm.at[pl.ds(base, ROWS_PER_W)], idx_dv,
                               sem_i),
              pltpu.async_copy(idx_o_hbm.at[pl.ds(base, ROWS_PER_W)], idx_ov,
                               sem_i)]
        for c in ci:
            c.wait()

        def fuse(k, carry):
            sl = pl.ds(pl.multiple_of(k * LANES, LANES), LANES)
            idx_fv[sl] = (idx_sv[sl] * (7 * 200) + idx_dv[sl] * 200) + idx_ov[sl]
            return carry

        lax.fori_loop(0, ROWS_PER_W // LANES, fuse, 0)

        gs = [pltpu.async_copy(ft_hbm.at[idx_fv.at[pl.ds(j * CHUNK, CHUNK)]],
                               comb.at[j], sem_g)
              for j in range(NCH_HBM)]
        st.wait()
        plsc.subcore_barrier()
        gs += [pltpu.async_copy(sft.at[idx_fv.at[pl.ds(j * CHUNK, CHUNK)]],
                                comb.at[j], sem_g)
               for j in range(NCH_HBM, NCH)]
        ws = []
        for j in range(NCH):
            gs[j].wait()
            ws.append(pltpu.async_copy(
                comb.at[j], out_hbm.at[pl.ds(base + j * CHUNK, CHUNK), :],
                sem_w))
        for w in ws:
            w.wait()

    return body(idx_s, idx_d, idx_o, fused_table)


@jax.jit
def kernel(scale_conditions, distortion_conditions, offset_conditions,
           scale_table, distortion_table, offset_table):
    idx_s = scale_conditions.astype(jnp.int32)
    idx_d = distortion_conditions.astype(jnp.int32)
    idx_o = offset_conditions.astype(jnp.int32)
    s_b = jnp.broadcast_to(scale_table[:, None, None, :], (2, 7, 200, PART))
    d_b = jnp.broadcast_to(distortion_table[None, :, None, :], (2, 7, 200, PART))
    o_b = jnp.broadcast_to(offset_table[None, None, :, :], (2, 7, 200, OFF_DIM))
    fused_table = jnp.concatenate([s_b, d_b, o_b], axis=-1).reshape(
        N_FUSED, EMB_DIM)
    fused_table = jnp.pad(fused_table, ((0, N_FUSED_PAD - N_FUSED), (0, 0)))
    return _sc_embed(idx_s, idx_d, idx_o, fused_table)
